# Initial kernel scaffold; baseline (speedup 1.0000x reference)
import jax
import jax.numpy as jnp
from jax.experimental import pallas as pl


def kernel(node_attr, edge_index, edge_attr, edge_sh, W1, b1, W2, b2):
    raise NotImplementedError("write your pallas kernel here")


# speedup vs baseline: 1.5291x; 1.5291x over previous
import functools

import jax
import jax.numpy as jnp
import numpy as np
from jax import lax
from jax.experimental import pallas as pl
from jax.experimental.pallas import tpu as pltpu
from jax.experimental.pallas import tpu_sc as plsc

N = 10000
E = 160000
NS = 16
NV = 4
F = 48
CH = 128
NCHUNK = E // CH
NCHUNK_PAD = 1280
NW = 32
EB = 2000
GRID = E // EB
INV = 0.25
NACC = 10240

_c = np.arange(256)
_R16 = np.zeros((16, 256), np.float32)
_R16[_c // 16, _c] = 1.0
_Q16 = np.zeros((256, 16), np.float32)
_Q16[_c, _c % 16] = INV
_c4 = np.arange(64)
_R4 = np.zeros((16, 64), np.float32)
_R4[_c4 // 4, _c4] = 1.0
_Q4 = np.zeros((64, 16), np.float32)
_Q4[_c4, _c4 % 4] = INV
_B0 = np.zeros((16, 16), np.float32)
_B0[0, :] = 1.0
_RT = np.zeros((16, 16), np.float32)
_RS = np.zeros((16, 16), np.float32)
for _j in range(NV):
    for _m in range(3):
        _RT[_j, 3 * _j + _m] = 1.0
        _RS[1 + _m, 3 * _j + _m] = 1.0

@functools.lru_cache(maxsize=1)
def _sc_mesh():
    return plsc.VectorSubcoreMesh(
        core_axis_name="c", subcore_axis_name="s", num_cores=2, num_subcores=16)


def _worker_range(w):
    base = pl.multiple_of(40 * w, 8)
    nch = jnp.where(w == 31, 10, 40)
    return base, nch


def _sc_gather(node_attr, dst2):
    @functools.partial(
        pl.kernel,
        out_type=jax.ShapeDtypeStruct((NCHUNK, CH, NS), jnp.float32),
        mesh=_sc_mesh(),
        compiler_params=pltpu.CompilerParams(use_tc_tiling_on_sc=False),
        scratch_types=[
            pltpu.VMEM((40, CH), jnp.int32),
            pltpu.VMEM((40, CH, NS), jnp.float32),
            pltpu.SemaphoreType.DMA,
        ],
    )
    def k(node_hbm, idx_hbm, out_hbm, idx_v, rows_v, sem):
        w = lax.axis_index("s") * 2 + lax.axis_index("c")
        base, nch = _worker_range(w)
        pltpu.sync_copy(idx_hbm.at[pl.ds(base, 40)], idx_v)

        def fire(j, carry):
            pltpu.async_copy(node_hbm.at[idx_v.at[j]], rows_v.at[j], sem)
            return carry

        lax.fori_loop(0, nch, fire, 0)

        def drain(j, carry):
            pltpu.make_async_copy(node_hbm.at[idx_v.at[j]], rows_v.at[j], sem).wait()
            return carry

        lax.fori_loop(0, nch, drain, 0)

        @pl.when(w < 31)
        def _():
            pltpu.sync_copy(rows_v, out_hbm.at[pl.ds(base, 40)])

        @pl.when(w == 31)
        def _():
            pltpu.sync_copy(rows_v.at[pl.ds(0, 10)], out_hbm.at[pl.ds(base, 10)])

    return k(node_attr, dst2)


def _tc_compute(ea, shp, xg, W1, b1, W2a, b2a, W2b, b2b):
    consts = (jnp.asarray(_B0), jnp.asarray(_R16), jnp.asarray(_Q16),
              jnp.asarray(_R4), jnp.asarray(_Q4), jnp.asarray(_RT),
              jnp.asarray(_RS))

    def body(ea_ref, shp_ref, xg_ref, W1_ref, b1_ref, W2a_ref, b2a_ref,
             W2b_ref, b2b_ref, B0_ref, R16_ref, Q16_ref, R4_ref, Q4_ref,
             RT_ref, RS_ref, o_ref):
        dot = lambda a, b: lax.dot(a, b, precision=lax.Precision.HIGHEST)
        mlp = lambda a, b: lax.dot(a, b)
        a = jnp.maximum(mlp(ea_ref[...], W1_ref[...]) + b1_ref[...], 0.0)
        h0 = mlp(a, W2a_ref[...]) + b2a_ref[...]
        h1 = mlp(a, W2b_ref[...]) + b2b_ref[...]
        xgv = xg_ref[...]
        shv = shp_ref[...]
        x0 = xgv * dot(shv, B0_ref[...])
        out0 = dot(dot(x0, R16_ref[...]) * h0, Q16_ref[...])
        t = dot(dot(xgv, R4_ref[...]) * h1, Q4_ref[...])
        lanes = lax.broadcasted_iota(jnp.int32, (EB, 16), 1)
        cnt = jnp.where(lanes == 12, 1.0, 0.0)
        out1p = dot(t, RT_ref[...]) * dot(shv, RS_ref[...]) + cnt
        o_ref[...] = jnp.concatenate([out0, out1p], axis=1)

    whole = lambda shape: pl.BlockSpec(shape, lambda i: (0,) * len(shape))
    row = lambda width: pl.BlockSpec((EB, width), lambda i: (i, 0))
    return pl.pallas_call(
        body,
        grid=(GRID,),
        in_specs=[
            row(F), row(NS), row(NS),
            whole((F, F)), whole((1, F)), whole((F, 256)), whole((1, 256)),
            whole((F, 64)), whole((1, 64)),
            whole((16, 16)), whole((16, 256)), whole((256, 16)),
            whole((16, 64)), whole((64, 16)), whole((16, 16)), whole((16, 16)),
        ],
        out_specs=pl.BlockSpec((EB, 32), lambda i: (i, 0)),
        out_shape=jax.ShapeDtypeStruct((E, 32), jnp.float32),
    )(ea, shp, xg, W1, b1, W2a, b2a, W2b, b2b, *consts)


def _sc_scatter(tp3, src2, zrows):
    @functools.partial(
        pl.kernel,
        out_type=jax.ShapeDtypeStruct((2, NACC, 32), jnp.float32),
        mesh=_sc_mesh(),
        compiler_params=pltpu.CompilerParams(use_tc_tiling_on_sc=False),
        scratch_types=[
            pltpu.VMEM((40, CH), jnp.int32),
            pltpu.VMEM((CH, 32), jnp.float32),
            pltpu.VMEM_SHARED((NACC, 32), jnp.float32),
            pltpu.SemaphoreType.DMA,
        ],
    )
    def k(tp_hbm, idx_hbm, z_hbm, out_hbm, idx_v, rows_v, acc, sem):
        cid = lax.axis_index("c")
        sid = lax.axis_index("s")
        w = sid * 2 + cid
        base, nch = _worker_range(w)
        arow = pl.multiple_of(sid * (NACC // 16), 8)
        pltpu.sync_copy(z_hbm, acc.at[pl.ds(arow, NACC // 16)])
        plsc.subcore_barrier()
        pltpu.sync_copy(idx_hbm.at[pl.ds(base, 40)], idx_v)

        def body(j, carry):
            pltpu.sync_copy(tp_hbm.at[base + j], rows_v)
            pltpu.sync_copy(rows_v, acc.at[idx_v.at[j]], add=True)
            return carry

        lax.fori_loop(0, nch, body, 0)
        plsc.subcore_barrier()
        pltpu.sync_copy(acc.at[pl.ds(arow, NACC // 16)],
                        out_hbm.at[cid, pl.ds(arow, NACC // 16)])

    return k(tp3, src2, zrows)


def _tc_combine(parts):
    def body(p_ref, o_ref):
        p = p_ref[...]
        s = p[0] + p[1]
        cnt = jnp.maximum(s[:N, 28:29], 1.0)
        o_ref[...] = s[:N, :28] / cnt

    return pl.pallas_call(
        body,
        in_specs=[pl.BlockSpec((2, NACC, 32), lambda: (0, 0, 0))],
        out_specs=pl.BlockSpec((N, 28), lambda: (0, 0)),
        out_shape=jax.ShapeDtypeStruct((N, 28), jnp.float32),
    )(parts)


def kernel(node_attr, edge_index, edge_attr, edge_sh, W1, b1, W2, b2):
    edge_src = edge_index[0]
    edge_dst = edge_index[1]
    pad = NCHUNK_PAD * CH - E
    dst2 = jnp.pad(edge_dst, (0, pad)).reshape(NCHUNK_PAD, CH)
    src2 = jnp.pad(edge_src, (0, pad)).reshape(NCHUNK_PAD, CH)
    shp = jnp.pad(edge_sh, ((0, 0), (0, NS - 9)))

    xg = _sc_gather(node_attr, dst2).reshape(E, NS)
    tp = _tc_compute(
        edge_attr, shp, xg, W1, b1.reshape(1, F),
        W2[:, :256], b2[:256].reshape(1, 256),
        W2[:, 256:], b2[256:].reshape(1, 64))
    parts = _sc_scatter(tp.reshape(NCHUNK, CH, 32), src2,
                        jnp.zeros((NACC // 16, 32), jnp.float32))
    return _tc_combine(parts)


# speedup vs baseline: 4.0640x; 2.6578x over previous
import functools

import jax
import jax.numpy as jnp
import numpy as np
from jax import lax
from jax.experimental import pallas as pl
from jax.experimental.pallas import tpu as pltpu
from jax.experimental.pallas import tpu_sc as plsc

N = 10000
E = 160000
NS = 16
NV = 4
F = 48
CH = 128
NCHUNK = E // CH
NCHUNK_PAD = 1280
NW = 32
EB = 4000
GRID = E // EB
INV = 0.25
NACC = 10240

_P0 = np.arange(256).reshape(16, 16).T.flatten()
_P1 = (4 * (np.arange(64) % 16) + np.arange(64) // 16)
_c = np.arange(256)
_Q16 = np.zeros((256, 16), np.float32)
_Q16[_c, _c // 16] = INV
_c4 = np.arange(64)
_Q4RT = np.zeros((64, 16), np.float32)
for _j in range(NV):
    for _i in range(NS):
        for _m in range(3):
            _Q4RT[16 * _j + _i, 3 * _j + _m] = INV
_TILE_IDX = np.arange(256) % 16
_SB = np.zeros((16, 32), np.float32)
_SB[0, :16] = 1.0
for _j in range(NV):
    for _m in range(3):
        _SB[1 + _m, 16 + 3 * _j + _m] = 1.0

@functools.lru_cache(maxsize=1)
def _sc_mesh():
    return plsc.VectorSubcoreMesh(
        core_axis_name="c", subcore_axis_name="s", num_cores=2, num_subcores=16)


def _worker_range(w):
    base = pl.multiple_of(40 * w, 8)
    nch = jnp.where(w == 31, 10, 40)
    return base, nch


def _sc_gather(node_attr, dst2):
    @functools.partial(
        pl.kernel,
        out_type=jax.ShapeDtypeStruct((NCHUNK, CH, NS), jnp.float32),
        mesh=_sc_mesh(),
        compiler_params=pltpu.CompilerParams(use_tc_tiling_on_sc=False),
        scratch_types=[
            pltpu.VMEM((40, CH), jnp.int32),
            pltpu.VMEM((40, CH, NS), jnp.float32),
            pltpu.SemaphoreType.DMA,
        ],
    )
    def k(node_hbm, idx_hbm, out_hbm, idx_v, rows_v, sem):
        w = lax.axis_index("s") * 2 + lax.axis_index("c")
        base, nch = _worker_range(w)
        pltpu.sync_copy(idx_hbm.at[pl.ds(base, 40)], idx_v)

        def fire(j, carry):
            pltpu.async_copy(node_hbm.at[idx_v.at[j]], rows_v.at[j], sem)
            return carry

        lax.fori_loop(0, nch, fire, 0)

        def drain(j, carry):
            pltpu.make_async_copy(node_hbm.at[idx_v.at[j]], rows_v.at[j], sem).wait()
            return carry

        lax.fori_loop(0, nch, drain, 0)

        @pl.when(w < 31)
        def _():
            pltpu.sync_copy(rows_v, out_hbm.at[pl.ds(base, 40)])

        @pl.when(w == 31)
        def _():
            pltpu.sync_copy(rows_v.at[pl.ds(0, 10)], out_hbm.at[pl.ds(base, 10)])

    return k(node_attr, dst2)


def _tc_compute(ea, shp, xg, W1, b1, W2a, b2a, W2b, b2b):
    consts = (jnp.asarray(_Q16), jnp.asarray(_Q4RT), jnp.asarray(_SB))

    def body(ea_ref, shp_ref, xg_ref, W1_ref, b1_ref, W2a_ref, b2a_ref,
             W2b_ref, b2b_ref, Q16_ref, Q4RT_ref, SB_ref, o_ref):
        dot = lambda a, b: lax.dot(a, b)

        a = jnp.maximum(dot(ea_ref[...], W1_ref[...]) + b1_ref[...], 0.0)
        h0p = dot(a, W2a_ref[...]) + b2a_ref[...]
        h1p = dot(a, W2b_ref[...]) + b2b_ref[...]
        xgv = xg_ref[...]
        shv = shp_ref[...]
        tidx = lax.broadcasted_iota(jnp.int32, (EB, 256), 1) % 16
        xt16 = jnp.take_along_axis(xgv, tidx, axis=1)
        pre0 = dot(xt16 * h0p, Q16_ref[...])
        tb = dot(xt16[:, :64] * h1p, Q4RT_ref[...])
        shall = dot(shv, SB_ref[...])
        lanes = lax.broadcasted_iota(jnp.int32, (EB, 32), 1)
        cnt = jnp.where(lanes == 28, 1.0, 0.0)
        o_ref[...] = jnp.concatenate([pre0, tb], axis=1) * shall + cnt

    whole = lambda shape: pl.BlockSpec(shape, lambda i: (0,) * len(shape))
    row = lambda width: pl.BlockSpec((EB, width), lambda i: (i, 0))
    return pl.pallas_call(
        body,
        grid=(GRID,),
        in_specs=[
            row(F), row(NS), row(NS),
            whole((F, F)), whole((1, F)), whole((F, 256)), whole((1, 256)),
            whole((F, 64)), whole((1, 64)),
            whole((256, 16)), whole((64, 16)), whole((16, 32)),
        ],
        out_specs=pl.BlockSpec((EB, 32), lambda i: (i, 0)),
        out_shape=jax.ShapeDtypeStruct((E, 32), jnp.float32),
    )(ea, shp, xg, W1, b1, W2a, b2a, W2b, b2b, *consts)


def _sc_scatter(tp3, src2, zrows):
    @functools.partial(
        pl.kernel,
        out_type=jax.ShapeDtypeStruct((2, NACC, 32), jnp.float32),
        mesh=_sc_mesh(),
        compiler_params=pltpu.CompilerParams(use_tc_tiling_on_sc=False),
        scratch_types=[
            pltpu.VMEM((40, CH), jnp.int32),
            pltpu.VMEM((CH, 32), jnp.float32),
            pltpu.VMEM_SHARED((NACC, 32), jnp.float32),
            pltpu.SemaphoreType.DMA,
        ],
    )
    def k(tp_hbm, idx_hbm, z_hbm, out_hbm, idx_v, rows_v, acc, sem):
        cid = lax.axis_index("c")
        sid = lax.axis_index("s")
        w = sid * 2 + cid
        base, nch = _worker_range(w)
        arow = pl.multiple_of(sid * (NACC // 16), 8)
        pltpu.sync_copy(z_hbm, acc.at[pl.ds(arow, NACC // 16)])
        plsc.subcore_barrier()
        pltpu.sync_copy(idx_hbm.at[pl.ds(base, 40)], idx_v)

        def body(j, carry):
            pltpu.sync_copy(tp_hbm.at[base + j], rows_v)
            pltpu.sync_copy(rows_v, acc.at[idx_v.at[j]], add=True)
            return carry

        lax.fori_loop(0, nch, body, 0)
        plsc.subcore_barrier()
        pltpu.sync_copy(acc.at[pl.ds(arow, NACC // 16)],
                        out_hbm.at[cid, pl.ds(arow, NACC // 16)])

    return k(tp3, src2, zrows)


def _tc_combine(parts):
    def body(p_ref, o_ref):
        p = p_ref[...]
        s = p[0] + p[1]
        cnt = jnp.maximum(s[:N, 28:29], 1.0)
        o_ref[...] = s[:N, :28] / cnt

    return pl.pallas_call(
        body,
        in_specs=[pl.BlockSpec((2, NACC, 32), lambda: (0, 0, 0))],
        out_specs=pl.BlockSpec((N, 28), lambda: (0, 0)),
        out_shape=jax.ShapeDtypeStruct((N, 28), jnp.float32),
    )(parts)


def kernel(node_attr, edge_index, edge_attr, edge_sh, W1, b1, W2, b2):
    edge_src = edge_index[0]
    edge_dst = edge_index[1]
    pad = NCHUNK_PAD * CH - E
    dst2 = jnp.pad(edge_dst, (0, pad)).reshape(NCHUNK_PAD, CH)
    src2 = jnp.pad(edge_src, (0, pad)).reshape(NCHUNK_PAD, CH)
    shp = jnp.pad(edge_sh, ((0, 0), (0, NS - 9)))

    xg = _sc_gather(node_attr, dst2).reshape(E, NS)
    W2a = W2[:, :256][:, _P0]
    b2a = b2[:256][_P0]
    W2b = W2[:, 256:][:, _P1]
    b2b = b2[256:][_P1]
    tp = _tc_compute(
        edge_attr, shp, xg, W1, b1.reshape(1, F),
        W2a, b2a.reshape(1, 256), W2b, b2b.reshape(1, 64))
    parts = _sc_scatter(tp.reshape(NCHUNK, CH, 32), src2,
                        jnp.zeros((NACC // 16, 32), jnp.float32))
    return _tc_combine(parts)


# speedup vs baseline: 4.1670x; 1.0253x over previous
import functools

import jax
import jax.numpy as jnp
import numpy as np
from jax import lax
from jax.experimental import pallas as pl
from jax.experimental.pallas import tpu as pltpu
from jax.experimental.pallas import tpu_sc as plsc

N = 10000
E = 160000
NS = 16
NV = 4
F = 48
CH = 128
NCHUNK = E // CH
NCHUNK_PAD = 1280
NW = 32
EB = 4000
GRID = E // EB
INV = 0.25
NACC = 10240

_P0 = np.arange(256).reshape(16, 16).T.flatten()
_P1 = (4 * (np.arange(64) % 16) + np.arange(64) // 16)
_c = np.arange(256)
_Q16 = np.zeros((256, 16), np.float32)
_Q16[_c, _c // 16] = INV
_c4 = np.arange(64)
_Q4RT = np.zeros((64, 16), np.float32)
for _j in range(NV):
    for _i in range(NS):
        for _m in range(3):
            _Q4RT[16 * _j + _i, 3 * _j + _m] = INV
_TILE_IDX = np.arange(256) % 16
_SB = np.zeros((16, 32), np.float32)
_SB[0, :16] = 1.0
for _j in range(NV):
    for _m in range(3):
        _SB[1 + _m, 16 + 3 * _j + _m] = 1.0

@functools.lru_cache(maxsize=1)
def _sc_mesh():
    return plsc.VectorSubcoreMesh(
        core_axis_name="c", subcore_axis_name="s", num_cores=2, num_subcores=16)


def _worker_range(w):
    base = pl.multiple_of(40 * w, 8)
    nch = jnp.where(w == 31, 10, 40)
    return base, nch


def _sc_gather(node_attr, dst2):
    @functools.partial(
        pl.kernel,
        out_type=jax.ShapeDtypeStruct((E, NS), jnp.float32),
        mesh=_sc_mesh(),
        compiler_params=pltpu.CompilerParams(use_tc_tiling_on_sc=False),
        scratch_types=[
            pltpu.VMEM((40, CH), jnp.int32),
            pltpu.VMEM((40 * CH, NS), jnp.float32),
            pltpu.SemaphoreType.DMA,
        ],
    )
    def k(node_hbm, idx_hbm, out_hbm, idx_v, rows_v, sem):
        w = lax.axis_index("s") * 2 + lax.axis_index("c")
        base, nch = _worker_range(w)
        pltpu.sync_copy(idx_hbm.at[pl.ds(base, 40)], idx_v)

        def fire(j, carry):
            r = pl.multiple_of(j * CH, 8)
            pltpu.async_copy(node_hbm.at[idx_v.at[j]],
                             rows_v.at[pl.ds(r, CH)], sem)
            return carry

        lax.fori_loop(0, nch, fire, 0)

        def drain(j, carry):
            r = pl.multiple_of(j * CH, 8)
            pltpu.make_async_copy(node_hbm.at[idx_v.at[j]],
                                  rows_v.at[pl.ds(r, CH)], sem).wait()
            return carry

        lax.fori_loop(0, nch, drain, 0)
        rbase = pl.multiple_of(base * CH, 8)

        @pl.when(w < 31)
        def _():
            pltpu.sync_copy(rows_v, out_hbm.at[pl.ds(rbase, 40 * CH)])

        @pl.when(w == 31)
        def _():
            pltpu.sync_copy(rows_v.at[pl.ds(0, 10 * CH)],
                            out_hbm.at[pl.ds(rbase, 10 * CH)])

    return k(node_attr, dst2)


def _tc_compute(ea, shp, xg, W1, b1, W2a, b2a, W2b, b2b):
    consts = (jnp.asarray(_Q16), jnp.asarray(_Q4RT), jnp.asarray(_SB[:9]))

    def body(ea_ref, shp_ref, xg_ref, W1_ref, b1_ref, W2a_ref, b2a_ref,
             W2b_ref, b2b_ref, Q16_ref, Q4RT_ref, SB_ref, o_ref):
        dot = lambda a, b: lax.dot(a, b)

        a = jnp.maximum(dot(ea_ref[...], W1_ref[...]) + b1_ref[...], 0.0)
        h0p = dot(a, W2a_ref[...]) + b2a_ref[...]
        h1p = dot(a, W2b_ref[...]) + b2b_ref[...]
        xgv = xg_ref[...]
        shv = shp_ref[...]
        tidx = lax.broadcasted_iota(jnp.int32, (EB, 256), 1) % 16
        xt16 = jnp.take_along_axis(xgv, tidx, axis=1)
        pre0 = dot(xt16 * h0p, Q16_ref[...])
        tb = dot(xt16[:, :64] * h1p, Q4RT_ref[...])
        shall = dot(shv, SB_ref[...])
        lanes = lax.broadcasted_iota(jnp.int32, (EB, 32), 1)
        cnt = jnp.where(lanes == 28, 1.0, 0.0)
        o_ref[...] = jnp.concatenate([pre0, tb], axis=1) * shall + cnt

    whole = lambda shape: pl.BlockSpec(shape, lambda i: (0,) * len(shape))
    row = lambda width: pl.BlockSpec((EB, width), lambda i: (i, 0))
    return pl.pallas_call(
        body,
        grid=(GRID,),
        in_specs=[
            row(F), row(9), row(NS),
            whole((F, F)), whole((1, F)), whole((F, 256)), whole((1, 256)),
            whole((F, 64)), whole((1, 64)),
            whole((256, 16)), whole((64, 16)), whole((9, 32)),
        ],
        out_specs=pl.BlockSpec((EB, 32), lambda i: (i, 0)),
        out_shape=jax.ShapeDtypeStruct((E, 32), jnp.float32),
    )(ea, shp, xg, W1, b1, W2a, b2a, W2b, b2b, *consts)


def _sc_scatter(tp3, src2, zrows):
    @functools.partial(
        pl.kernel,
        out_type=jax.ShapeDtypeStruct((2, NACC, 32), jnp.float32),
        mesh=_sc_mesh(),
        compiler_params=pltpu.CompilerParams(use_tc_tiling_on_sc=False),
        scratch_types=[
            pltpu.VMEM((40, CH), jnp.int32),
            pltpu.VMEM((CH, 32), jnp.float32),
            pltpu.VMEM_SHARED((NACC, 32), jnp.float32),
            pltpu.SemaphoreType.DMA,
        ],
    )
    def k(tp_hbm, idx_hbm, z_hbm, out_hbm, idx_v, rows_v, acc, sem):
        cid = lax.axis_index("c")
        sid = lax.axis_index("s")
        w = sid * 2 + cid
        base, nch = _worker_range(w)
        arow = pl.multiple_of(sid * (NACC // 16), 8)
        pltpu.sync_copy(z_hbm, acc.at[pl.ds(arow, NACC // 16)])
        plsc.subcore_barrier()
        pltpu.sync_copy(idx_hbm.at[pl.ds(base, 40)], idx_v)

        def body(j, carry):
            r = pl.multiple_of((base + j) * CH, 8)
            pltpu.sync_copy(tp_hbm.at[pl.ds(r, CH)], rows_v)
            pltpu.sync_copy(rows_v, acc.at[idx_v.at[j]], add=True)
            return carry

        lax.fori_loop(0, nch, body, 0)
        plsc.subcore_barrier()
        pltpu.sync_copy(acc.at[pl.ds(arow, NACC // 16)],
                        out_hbm.at[cid, pl.ds(arow, NACC // 16)])

    return k(tp3, src2, zrows)


def _tc_combine(parts):
    def body(p_ref, o_ref):
        p = p_ref[...]
        s = p[0] + p[1]
        cnt = jnp.maximum(s[:N, 28:29], 1.0)
        o_ref[...] = s[:N, :28] / cnt

    return pl.pallas_call(
        body,
        in_specs=[pl.BlockSpec((2, NACC, 32), lambda: (0, 0, 0))],
        out_specs=pl.BlockSpec((N, 28), lambda: (0, 0)),
        out_shape=jax.ShapeDtypeStruct((N, 28), jnp.float32),
    )(parts)


def kernel(node_attr, edge_index, edge_attr, edge_sh, W1, b1, W2, b2):
    edge_src = edge_index[0]
    edge_dst = edge_index[1]
    pad = NCHUNK_PAD * CH - E
    dst2 = jnp.pad(edge_dst, (0, pad)).reshape(NCHUNK_PAD, CH)
    src2 = jnp.pad(edge_src, (0, pad)).reshape(NCHUNK_PAD, CH)
    xg = _sc_gather(node_attr, dst2)
    W2a = W2[:, :256][:, _P0]
    b2a = b2[:256][_P0]
    W2b = W2[:, 256:][:, _P1]
    b2b = b2[256:][_P1]
    tp = _tc_compute(
        edge_attr, edge_sh, xg, W1, b1.reshape(1, F),
        W2a, b2a.reshape(1, 256), W2b, b2b.reshape(1, 64))
    parts = _sc_scatter(tp, src2,
                        jnp.zeros((NACC // 16, 32), jnp.float32))
    return _tc_combine(parts)
